# baseline (speedup 1.0000x reference)
import functools

import numpy as np
import jax
import jax.numpy as jnp
from jax import lax
from jax.experimental import pallas as pl
from jax.experimental.pallas import tpu as pltpu
from jax.experimental.pallas import tpu_sc as plsc

L = 32
LAT = L * L
B = 1024
TAIL = 16
NW = 32
NB = B // NW
SPREAD = L
SPREAD_ROWS = TAIL * SPREAD
GS = 1
NG = NB // GS
GROUP_ROWS = GS * SPREAD_ROWS
GROUP_SITES = GS * LAT


def _build_flag_matrix() -> np.ndarray:
    lp = np.zeros((L, L), dtype=np.int64)
    for m in range(L):
        v = np.zeros(L, dtype=np.int64)
        v[m] = 1
        lp[:, m] = np.roll(np.cumsum(np.roll(v[::-1], 1)), 1)
    w = np.zeros((2 * LAT, 4 * L), dtype=np.int64)
    specs = [(0, 0, 1), (1, 0, 0), (0, 1, 1), (1, 1, 0)]
    for f, (half, axis, shift) in enumerate(specs):
        a = np.zeros((L, LAT), dtype=np.int64)
        for k in range(L):
            if axis == 0:
                a[k, np.arange(L) * L + (k - shift) % L] = 1
            else:
                a[k, ((k - shift) % L) * L + np.arange(L)] = 1
        w[half * LAT:(half + 1) * LAT, f * L:(f + 1) * L] = (lp @ a).T
    return w


def _build_xor_masks() -> list[int]:
    def primal_src(axis):
        t = np.arange(TAIL).reshape(4, 2, 2)
        return np.roll(t, 1, axis=2 - axis).reshape(TAIL)

    comm = np.array([0, 2, 1, 3])

    def dual_tf(y16):
        y = y16.reshape(4, 2, 2)
        y = np.transpose(y, (2, 1, 0)).reshape(4, 2, 2)
        return y[comm, :, :].reshape(TAIL)

    def dual_src(axis):
        c = dual_tf(np.arange(TAIL))
        c = np.roll(c.reshape(4, 2, 2), 1, axis=1 + axis).reshape(TAIL)
        return dual_tf(c)

    ident = np.arange(TAIL)
    passes = {}
    for axis in range(2):
        pp, dp = primal_src(axis), dual_src(axis)
        for fp in range(2):
            for fd in range(2):
                s = ident
                if fp:
                    s = s[pp]
                if fd:
                    s = s[dp]
                passes[(axis, fp, fd)] = s
    g = []
    for cls in range(TAIL):
        r, c = cls // 4, cls % 4
        s0 = passes[(0, c // 2, c % 2)]
        s1 = passes[(1, r // 2, r % 2)]
        src = s0[s1]
        assert np.all((ident ^ src[0]) == src), f"class {cls} not an XOR mask"
        g.append(int(src[0]))
    return g


_W_NP = _build_flag_matrix().astype(np.float32)
_G = _build_xor_masks()


def _flag_body(syn_ref, w_ref, rc_ref):
    syn = syn_ref[...].astype(jnp.bfloat16)
    fv = jnp.dot(syn, w_ref[...], preferred_element_type=jnp.float32)
    bit = jnp.bitwise_and(fv.astype(jnp.int32), 1)
    fp0 = bit[:, 0:32]
    fd0 = bit[:, 32:64]
    fp1 = bit[:, 64:96]
    fd1 = bit[:, 96:128]
    rbase = (2 * fp1 + fd1) * 128
    cpart = (2 * fp0 + fd0) * L + jnp.broadcast_to(
        lax.broadcasted_iota(jnp.int32, (1, L), 1), fp0.shape)
    rc_ref[...] = jnp.concatenate([rbase, cpart], axis=-1)


def _class_codes(syndrome):
    blk = 256
    return pl.pallas_call(
        _flag_body,
        grid=(B // blk,),
        in_specs=[
            pl.BlockSpec((blk, 2 * LAT), lambda i: (i, 0)),
            pl.BlockSpec((2 * LAT, 4 * L), lambda i: (0, 0)),
        ],
        out_specs=pl.BlockSpec((blk, 2 * L), lambda i: (i, 0)),
        out_shape=jax.ShapeDtypeStruct((B, 2 * L), jnp.int32),
    )(syndrome, jnp.asarray(_W_NP, dtype=jnp.bfloat16))


@functools.cache
def _get_sc_pool():
    return pl.kernel(
        _sc_pool_body,
        out_type=jax.ShapeDtypeStruct((B, TAIL), jnp.float32),
        mesh=plsc.VectorSubcoreMesh(core_axis_name="c", subcore_axis_name="s"),
        compiler_params=pltpu.CompilerParams(needs_layout_passes=False,
                                             use_tc_tiling_on_sc=False),
        scratch_types=[
        pltpu.VMEM((GROUP_SITES, TAIL), jnp.float32),
        pltpu.VMEM((GROUP_SITES, TAIL), jnp.float32),
        pltpu.VMEM((GROUP_SITES,), jnp.int32),
        pltpu.VMEM((GROUP_SITES,), jnp.int32),
        pltpu.VMEM((NB, 2 * L), jnp.int32),
        pltpu.VMEM((GROUP_ROWS, TAIL), jnp.float32),
        pltpu.VMEM((GROUP_ROWS, TAIL), jnp.float32),
        pltpu.VMEM((GS * TAIL, TAIL), jnp.float32),
        pltpu.VMEM((NB, TAIL), jnp.float32),
        pltpu.VMEM_SHARED((2 * 16 * GROUP_ROWS, TAIL), jnp.float32),
        pltpu.SemaphoreType.DMA,
        pltpu.SemaphoreType.DMA,
        pltpu.SemaphoreType.DMA,
        ],
    )


def _sc_pool_body(x_hbm, rc_hbm, out_hbm, xbuf0, xbuf1, idx0, idx1, rcbuf,
                  accbuf, zbuf, sumbuf, outbuf, acc_sh, semx, semsc, semz):
    cid = lax.axis_index("c")
    sid = lax.axis_index("s")
    wid = sid * 2 + cid
    base = wid * NB
    xrow = base * LAT
    srow_a = sid * GROUP_ROWS
    srow_b = (16 + sid) * GROUP_ROWS

    pltpu.sync_copy(rc_hbm.at[pl.ds(base, NB)], rcbuf)

    zeros16 = jnp.zeros((TAIL,), jnp.float32)
    iota16 = lax.iota(jnp.int32, TAIL)

    def zero_rows(rr, carry):
        zbuf[rr, :] = zeros16
        return carry

    lax.fori_loop(0, GROUP_ROWS, zero_rows, 0)

    def zero_region(sbase):
        pltpu.async_copy(zbuf, acc_sh.at[pl.ds(sbase, GROUP_ROWS)], semz)

    def wait_zero(sbase):
        pltpu.make_async_copy(zbuf, acc_sh.at[pl.ds(sbase, GROUP_ROWS)],
                              semz).wait()

    def build_idx(idx_ref, g, sbase):
        for t in range(GS):
            k = g * GS + t
            off = sbase + t * SPREAD_ROWS
            cvec0 = rcbuf[k, pl.ds(L, TAIL)] + off
            cvec1 = rcbuf[k, pl.ds(L + TAIL, TAIL)] + off
            for hi in range(2):
                rvec = rcbuf[k, pl.ds(hi * TAIL, TAIL)]
                for ii in range(TAIL):
                    i = hi * TAIL + ii
                    rr = rvec[ii]
                    idx_ref[pl.ds(t * LAT + i * L, TAIL)] = rr + cvec0
                    idx_ref[pl.ds(t * LAT + i * L + TAIL, TAIL)] = rr + cvec1

    def fetch(g, xb):
        pltpu.async_copy(x_hbm.at[pl.ds(xrow + g * GROUP_SITES, GROUP_SITES)],
                         xb, semx)

    def wait_fetch(g, xb):
        pltpu.make_async_copy(
            x_hbm.at[pl.ds(xrow + g * GROUP_SITES, GROUP_SITES)], xb,
            semx).wait()

    def drain(g, sbase, do_zero=True):
        pltpu.sync_copy(acc_sh.at[pl.ds(sbase, GROUP_ROWS)], accbuf)
        if do_zero:
            zero_region(sbase)

        for kk in range(GS * TAIL):
            s = accbuf[kk * SPREAD, :]
            for m in range(1, SPREAD):
                s = s + accbuf[kk * SPREAD + m, :]
            sumbuf[kk, :] = s
        for t in range(GS):
            o = zeros16
            for cls in range(TAIL):
                lanes = jnp.bitwise_xor(iota16, _G[cls])
                rows = jnp.full((TAIL,), t * TAIL + cls, jnp.int32)
                o = o + plsc.load_gather(sumbuf, [rows, lanes])
            outbuf[g * GS + t, :] = o * jnp.float32(1.0 / LAT)

    fetch(0, xbuf0)
    build_idx(idx0, 0, srow_a)
    zero_region(srow_a)
    zero_region(srow_b)
    wait_zero(srow_a)
    wait_zero(srow_b)

    def pair(m, carry):
        g = 2 * m
        wait_fetch(g, xbuf0)

        @pl.when(m > 0)
        def _wait_zero_a():
            wait_zero(srow_a)

        scat_a = pltpu.async_copy(xbuf0, acc_sh.at[idx0], semsc, add=True)
        fetch(g + 1, xbuf1)
        build_idx(idx1, g + 1, srow_b)
        scat_a.wait()

        @pl.when(m > 0)
        def _drain_prev():
            drain(g - 1, srow_b)

        wait_fetch(g + 1, xbuf1)

        @pl.when(m > 0)
        def _wait_zero_b():
            wait_zero(srow_b)

        scat_b = pltpu.async_copy(xbuf1, acc_sh.at[idx1], semsc, add=True)

        @pl.when(g + 2 < NG)
        def _prefetch_next():
            fetch(g + 2, xbuf0)
            build_idx(idx0, g + 2, srow_a)

        scat_b.wait()
        drain(g, srow_a)
        return carry

    lax.fori_loop(0, NG // 2, pair, 0)
    wait_zero(srow_a)
    drain(NG - 1, srow_b, do_zero=False)
    pltpu.sync_copy(outbuf, out_hbm.at[pl.ds(base, NB)])


def kernel(x, syndrome):
    rc = _class_codes(syndrome)
    xs = x.reshape(B * LAT, TAIL)
    out = _get_sc_pool()(xs, rc)
    return out.reshape(B, 4, 2, 2)


# speedup vs baseline: 1.1948x; 1.1948x over previous
import functools

import numpy as np
import jax
import jax.numpy as jnp
from jax import lax
from jax.experimental import pallas as pl
from jax.experimental.pallas import tpu as pltpu
from jax.experimental.pallas import tpu_sc as plsc

L = 32
LAT = L * L
B = 1024
TAIL = 16
NW = 32
NB = B // NW
SPREAD = L
SPREAD_ROWS = TAIL * SPREAD
GS = 1
NG = NB // GS
GROUP_ROWS = GS * SPREAD_ROWS
GROUP_SITES = GS * LAT


def _build_flag_matrix() -> np.ndarray:
    lp = np.zeros((L, L), dtype=np.int64)
    for m in range(L):
        v = np.zeros(L, dtype=np.int64)
        v[m] = 1
        lp[:, m] = np.roll(np.cumsum(np.roll(v[::-1], 1)), 1)
    w = np.zeros((2 * LAT, 4 * L), dtype=np.int64)
    specs = [(0, 0, 1), (1, 0, 0), (0, 1, 1), (1, 1, 0)]
    for f, (half, axis, shift) in enumerate(specs):
        a = np.zeros((L, LAT), dtype=np.int64)
        for k in range(L):
            if axis == 0:
                a[k, np.arange(L) * L + (k - shift) % L] = 1
            else:
                a[k, ((k - shift) % L) * L + np.arange(L)] = 1
        w[half * LAT:(half + 1) * LAT, f * L:(f + 1) * L] = (lp @ a).T
    return w


def _build_xor_masks() -> list[int]:
    def primal_src(axis):
        t = np.arange(TAIL).reshape(4, 2, 2)
        return np.roll(t, 1, axis=2 - axis).reshape(TAIL)

    comm = np.array([0, 2, 1, 3])

    def dual_tf(y16):
        y = y16.reshape(4, 2, 2)
        y = np.transpose(y, (2, 1, 0)).reshape(4, 2, 2)
        return y[comm, :, :].reshape(TAIL)

    def dual_src(axis):
        c = dual_tf(np.arange(TAIL))
        c = np.roll(c.reshape(4, 2, 2), 1, axis=1 + axis).reshape(TAIL)
        return dual_tf(c)

    ident = np.arange(TAIL)
    passes = {}
    for axis in range(2):
        pp, dp = primal_src(axis), dual_src(axis)
        for fp in range(2):
            for fd in range(2):
                s = ident
                if fp:
                    s = s[pp]
                if fd:
                    s = s[dp]
                passes[(axis, fp, fd)] = s
    g = []
    for cls in range(TAIL):
        r, c = cls // 4, cls % 4
        s0 = passes[(0, c // 2, c % 2)]
        s1 = passes[(1, r // 2, r % 2)]
        src = s0[s1]
        assert np.all((ident ^ src[0]) == src), f"class {cls} not an XOR mask"
        g.append(int(src[0]))
    return g


_W_NP = _build_flag_matrix().astype(np.float32)
_G = _build_xor_masks()


def _flag_body(syn_ref, w_ref, rc_ref):
    syn = syn_ref[...].astype(jnp.bfloat16)
    fv = jnp.dot(syn, w_ref[...], preferred_element_type=jnp.float32)
    bit = jnp.bitwise_and(fv.astype(jnp.int32), 1)
    fp0 = bit[:, 0:32]
    fd0 = bit[:, 32:64]
    fp1 = bit[:, 64:96]
    fd1 = bit[:, 96:128]
    rbase = (2 * fp1 + fd1) * 128
    cpart = (2 * fp0 + fd0) * L + jnp.broadcast_to(
        lax.broadcasted_iota(jnp.int32, (1, L), 1), fp0.shape)
    rc_ref[...] = jnp.concatenate([rbase, cpart], axis=-1)


def _class_codes(syndrome):
    blk = 256
    return pl.pallas_call(
        _flag_body,
        grid=(B // blk,),
        in_specs=[
            pl.BlockSpec((blk, 2 * LAT), lambda i: (i, 0)),
            pl.BlockSpec((2 * LAT, 4 * L), lambda i: (0, 0)),
        ],
        out_specs=pl.BlockSpec((blk, 2 * L), lambda i: (i, 0)),
        out_shape=jax.ShapeDtypeStruct((B, 2 * L), jnp.int32),
    )(syndrome, jnp.asarray(_W_NP, dtype=jnp.bfloat16))


@functools.cache
def _get_sc_pool():
    return pl.kernel(
        _sc_pool_body,
        out_type=jax.ShapeDtypeStruct((B, TAIL), jnp.float32),
        mesh=plsc.VectorSubcoreMesh(core_axis_name="c", subcore_axis_name="s"),
        compiler_params=pltpu.CompilerParams(needs_layout_passes=False,
                                             use_tc_tiling_on_sc=False),
        scratch_types=[
        pltpu.VMEM((GROUP_SITES, TAIL), jnp.float32),
        pltpu.VMEM((GROUP_SITES, TAIL), jnp.float32),
        pltpu.VMEM((GROUP_SITES,), jnp.int32),
        pltpu.VMEM((GROUP_SITES,), jnp.int32),
        pltpu.VMEM((NB, 2 * L), jnp.int32),
        pltpu.VMEM((GROUP_ROWS, TAIL), jnp.float32),
        pltpu.VMEM((GROUP_ROWS, TAIL), jnp.float32),
        pltpu.VMEM((GS * TAIL, TAIL), jnp.float32),
        pltpu.VMEM((NB, TAIL), jnp.float32),
        pltpu.VMEM_SHARED((2 * 16 * GROUP_ROWS, TAIL), jnp.float32),
        pltpu.SemaphoreType.DMA,
        pltpu.SemaphoreType.DMA,
        pltpu.SemaphoreType.DMA,
        ],
    )


def _sc_pool_body(x_hbm, rc_hbm, out_hbm, xbuf0, xbuf1, idx0, idx1, rcbuf,
                  accbuf, zbuf, sumbuf, outbuf, acc_sh, semx, semsc, semz):
    cid = lax.axis_index("c")
    sid = lax.axis_index("s")
    wid = sid * 2 + cid
    base = wid * NB
    xrow = base * LAT
    srow_a = sid * GROUP_ROWS
    srow_b = (16 + sid) * GROUP_ROWS

    pltpu.sync_copy(rc_hbm.at[pl.ds(base, NB)], rcbuf)

    zeros16 = jnp.zeros((TAIL,), jnp.float32)
    iota16 = lax.iota(jnp.int32, TAIL)

    def zero_rows(rr, carry):
        zbuf[rr, :] = zeros16
        return carry

    lax.fori_loop(0, GROUP_ROWS, zero_rows, 0)

    def zero_region(sbase):
        pltpu.async_copy(zbuf, acc_sh.at[pl.ds(sbase, GROUP_ROWS)], semz)

    def wait_zero(sbase):
        pltpu.make_async_copy(zbuf, acc_sh.at[pl.ds(sbase, GROUP_ROWS)],
                              semz).wait()

    def build_idx(idx_ref, g, sbase):
        for t in range(GS):
            k = g * GS + t
            off = sbase + t * SPREAD_ROWS
            cvec0 = rcbuf[k, pl.ds(L, TAIL)] + off
            cvec1 = rcbuf[k, pl.ds(L + TAIL, TAIL)] + off
            for hi in range(2):
                rvec = rcbuf[k, pl.ds(hi * TAIL, TAIL)]
                for ii in range(TAIL):
                    i = hi * TAIL + ii
                    rr = rvec[ii]
                    idx_ref[pl.ds(t * LAT + i * L, TAIL)] = rr + cvec0
                    idx_ref[pl.ds(t * LAT + i * L + TAIL, TAIL)] = rr + cvec1

    def fetch(g, xb):
        pltpu.async_copy(x_hbm.at[base + g], xb, semx)

    def wait_fetch(g, xb):
        pltpu.make_async_copy(x_hbm.at[base + g], xb, semx).wait()

    def drain(g, sbase, do_zero=True):
        pltpu.sync_copy(acc_sh.at[pl.ds(sbase, GROUP_ROWS)], accbuf)
        if do_zero:
            zero_region(sbase)

        for kk in range(GS * TAIL):
            s = accbuf[kk * SPREAD, :]
            for m in range(1, SPREAD):
                s = s + accbuf[kk * SPREAD + m, :]
            sumbuf[kk, :] = s
        for t in range(GS):
            o = zeros16
            for cls in range(TAIL):
                lanes = jnp.bitwise_xor(iota16, _G[cls])
                rows = jnp.full((TAIL,), t * TAIL + cls, jnp.int32)
                o = o + plsc.load_gather(sumbuf, [rows, lanes])
            outbuf[g * GS + t, :] = o * jnp.float32(1.0 / LAT)

    fetch(0, xbuf0)
    build_idx(idx0, 0, srow_a)
    zero_region(srow_a)
    zero_region(srow_b)
    wait_zero(srow_a)
    wait_zero(srow_b)

    def pair(m, carry):
        g = 2 * m
        wait_fetch(g, xbuf0)

        @pl.when(m > 0)
        def _wait_zero_a():
            wait_zero(srow_a)

        scat_a = pltpu.async_copy(xbuf0, acc_sh.at[idx0], semsc, add=True)
        fetch(g + 1, xbuf1)
        build_idx(idx1, g + 1, srow_b)
        scat_a.wait()

        @pl.when(m > 0)
        def _drain_prev():
            drain(g - 1, srow_b)

        wait_fetch(g + 1, xbuf1)

        @pl.when(m > 0)
        def _wait_zero_b():
            wait_zero(srow_b)

        scat_b = pltpu.async_copy(xbuf1, acc_sh.at[idx1], semsc, add=True)

        @pl.when(g + 2 < NG)
        def _prefetch_next():
            fetch(g + 2, xbuf0)
            build_idx(idx0, g + 2, srow_a)

        scat_b.wait()
        drain(g, srow_a)
        return carry

    lax.fori_loop(0, NG // 2, pair, 0)
    wait_zero(srow_a)
    drain(NG - 1, srow_b, do_zero=False)
    pltpu.sync_copy(outbuf, out_hbm.at[pl.ds(base, NB)])


def kernel(x, syndrome):
    rc = _class_codes(syndrome)
    xs = x.reshape(B, LAT, TAIL)
    out = _get_sc_pool()(xs, rc)
    return out.reshape(B, 4, 2, 2)
